# Initial kernel scaffold; baseline (speedup 1.0000x reference)
import jax
import jax.numpy as jnp
from jax.experimental import pallas as pl


def kernel(x, edge_index, memory, W_ih, W_hh, b_ih, b_hh, W_t, b_t, b_lin):
    raise NotImplementedError("write your pallas kernel here")


# speedup vs baseline: 3.0835x; 3.0835x over previous
import functools

import jax
import jax.numpy as jnp
from jax import lax
from jax.experimental import pallas as pl
from jax.experimental.pallas import tpu as pltpu
from jax.experimental.pallas import tpu_sc as plsc

N = 10000
E = 160000
D = 256
HALF = 128
NC = 2
NS = 16
K = 80
EDGES_PER_TILE = E // NS
ROWS_PER_TILE = 624
ROWS_TAIL = N - NS * ROWS_PER_TILE
NUM_WINDOWS = EDGES_PER_TILE // K


def _sc_agg_body(xh_hbm, src_hbm, dst_hbm, out_hbm, src_v, dst_v, rows_v,
                 acc, sem):
    c = lax.axis_index("c")
    s = lax.axis_index("s")

    base_r = s * ROWS_PER_TILE
    pltpu.sync_copy(xh_hbm.at[c].at[pl.ds(base_r, ROWS_PER_TILE)],
                    acc.at[pl.ds(base_r, ROWS_PER_TILE)])

    @pl.when(s == NS - 1)
    def _():
        pltpu.sync_copy(
            xh_hbm.at[c].at[pl.ds(NS * ROWS_PER_TILE, ROWS_TAIL)],
            acc.at[pl.ds(NS * ROWS_PER_TILE, ROWS_TAIL)])

    plsc.subcore_barrier()

    base_e = s * EDGES_PER_TILE
    table = xh_hbm.at[c]

    def body(j, carry):
        off = base_e + j * K
        pltpu.sync_copy(src_hbm.at[pl.ds(off, K)], src_v)
        pltpu.sync_copy(dst_hbm.at[pl.ds(off, K)], dst_v)
        pltpu.async_copy(table.at[src_v], rows_v, sem).wait()
        pltpu.sync_copy(rows_v, acc.at[dst_v], add=True)
        return carry

    lax.fori_loop(0, NUM_WINDOWS, body, 0)
    plsc.subcore_barrier()

    pltpu.sync_copy(acc.at[pl.ds(base_r, ROWS_PER_TILE)],
                    out_hbm.at[c].at[pl.ds(base_r, ROWS_PER_TILE)])

    @pl.when(s == NS - 1)
    def _():
        pltpu.sync_copy(
            acc.at[pl.ds(NS * ROWS_PER_TILE, ROWS_TAIL)],
            out_hbm.at[c].at[pl.ds(NS * ROWS_PER_TILE, ROWS_TAIL)])


_sc_agg = pl.kernel(
    _sc_agg_body,
    out_type=jax.ShapeDtypeStruct((NC, N, HALF), jnp.float32),
    mesh=plsc.VectorSubcoreMesh(core_axis_name="c", subcore_axis_name="s"),
    scratch_types=[
        pltpu.VMEM((K,), jnp.int32),
        pltpu.VMEM((K,), jnp.int32),
        pltpu.VMEM((K, HALF), jnp.float32),
        pltpu.VMEM_SHARED((N, HALF), jnp.float32),
        pltpu.SemaphoreType.DMA,
    ],
)


def _tc_weight_body(mem_ref, wihT_ref, bi_ref, bh_ref, wt_ref, bt_ref,
                    out_ref):
    xt = mem_ref[...]
    gi_r = jnp.dot(xt, wihT_ref[0], preferred_element_type=jnp.float32)
    gi_z = jnp.dot(xt, wihT_ref[1], preferred_element_type=jnp.float32)
    gi_n = jnp.dot(xt, wihT_ref[2], preferred_element_type=jnp.float32)
    r = jax.nn.sigmoid(gi_r + bi_ref[0:1, :] + bh_ref[0:1, :])
    z = jax.nn.sigmoid(gi_z + bi_ref[1:2, :] + bh_ref[1:2, :])
    n = jnp.tanh(gi_n + bi_ref[2:3, :] + r * bh_ref[2:3, :])
    u = (1.0 - z) * n
    nw = jnp.sum(wt_ref[...] * u, axis=1, keepdims=True)
    out_ref[...] = nw + bt_ref[...]


def _tc_out_body(h_ref, wT_ref, b_ref, out_ref):
    acc = jnp.dot(h_ref[0], wT_ref[0:HALF, :],
                  preferred_element_type=jnp.float32)
    acc = acc + jnp.dot(h_ref[1], wT_ref[HALF:D, :],
                        preferred_element_type=jnp.float32)
    out_ref[...] = jnp.maximum(acc + b_ref[...], 0.0)


def kernel(x, edge_index, memory, W_ih, W_hh, b_ih, b_hh, W_t, b_t, b_lin):
    del W_hh

    xh = x.reshape(N, NC, HALF).transpose(1, 0, 2)
    src = edge_index[0]
    dst = edge_index[1]
    wihT = W_ih.reshape(3, D, D).transpose(0, 2, 1)
    bi = b_ih.reshape(3, D)
    bh = b_hh.reshape(3, D)
    bt = b_t.reshape(D * D, 1)

    h2 = _sc_agg(xh, src, dst)

    wt_blk = 8192
    n_wt = (D * D) // wt_blk
    new_w = pl.pallas_call(
        _tc_weight_body,
        grid=(n_wt,),
        in_specs=[
            pl.BlockSpec((1, D), lambda k: (0, 0)),
            pl.BlockSpec((3, D, D), lambda k: (0, 0, 0)),
            pl.BlockSpec((3, D), lambda k: (0, 0)),
            pl.BlockSpec((3, D), lambda k: (0, 0)),
            pl.BlockSpec((wt_blk, D), lambda k: (k, 0)),
            pl.BlockSpec((wt_blk, 1), lambda k: (k, 0)),
        ],
        out_specs=pl.BlockSpec((wt_blk, 1), lambda k: (k, 0)),
        out_shape=jax.ShapeDtypeStruct((D * D, 1), jnp.float32),
    )(memory, wihT, bi, bh, W_t, bt)

    wT = new_w.reshape(D, D).T

    row_blk = 1000
    out = pl.pallas_call(
        _tc_out_body,
        grid=(N // row_blk,),
        in_specs=[
            pl.BlockSpec((NC, row_blk, HALF), lambda i: (0, i, 0)),
            pl.BlockSpec((D, D), lambda i: (0, 0)),
            pl.BlockSpec((1, D), lambda i: (0, 0)),
        ],
        out_specs=pl.BlockSpec((row_blk, D), lambda i: (i, 0)),
        out_shape=jax.ShapeDtypeStruct((N, D), jnp.float32),
    )(h2, wT, b_lin.reshape(1, D))
    return out


# speedup vs baseline: 5.8463x; 1.8960x over previous
import jax
import jax.numpy as jnp
from jax import lax
from jax.experimental import pallas as pl
from jax.experimental.pallas import tpu as pltpu
from jax.experimental.pallas import tpu_sc as plsc

N = 10000
E = 160000
D = 256
HALF = 128
NC = 2
NS = 16
K = 80
W = 126
NPAD = 16
NA = N + NPAD
ROWS_PER_TILE = 624
ROWS_TAIL = NA - NS * ROWS_PER_TILE
NRB = 3
NIB = 6


def _sc_agg_body(xh_hbm, e4_hbm, out_hbm, idx_v, rows_v, acc,
                 sem_i, sem_g, sem_s):
    c = lax.axis_index("c")
    s = lax.axis_index("s")

    base_r = s * ROWS_PER_TILE
    pltpu.sync_copy(xh_hbm.at[c].at[pl.ds(base_r, ROWS_PER_TILE)],
                    acc.at[pl.ds(base_r, ROWS_PER_TILE)])

    @pl.when(s == NS - 1)
    def _():
        pltpu.sync_copy(
            xh_hbm.at[c].at[pl.ds(NS * ROWS_PER_TILE, ROWS_TAIL)],
            acc.at[pl.ds(NS * ROWS_PER_TILE, ROWS_TAIL)])

    plsc.subcore_barrier()

    table = xh_hbm.at[c]
    e4 = e4_hbm.at[s]

    def fire_idx(j, si):
        return pltpu.async_copy(e4.at[j], idx_v.at[si], sem_i.at[si])

    def wait_idx(j, si):
        pltpu.make_async_copy(e4.at[j], idx_v.at[si], sem_i.at[si]).wait()

    def fire_gather(b, si):
        return pltpu.async_copy(table.at[idx_v.at[si, 0]], rows_v.at[b],
                                sem_g.at[b])

    def wait_gather(b, si):
        pltpu.make_async_copy(table.at[idx_v.at[si, 0]], rows_v.at[b],
                              sem_g.at[b]).wait()

    def fire_scatter(b, si):
        return pltpu.async_copy(rows_v.at[b], acc.at[idx_v.at[si, 1]],
                                sem_s.at[b], add=True)

    def wait_scatter(b, si):
        pltpu.make_async_copy(rows_v.at[b], acc.at[idx_v.at[si, 1]],
                              sem_s.at[b]).wait()

    fire_idx(0, 0)
    fire_idx(1, 1)
    wait_idx(0, 0)
    fire_gather(0, 0)
    fire_idx(2, 2)
    wait_idx(1, 1)
    fire_gather(1, 1)
    fire_idx(3, 3)
    for o in range(2, 6):
        if o >= 3:
            wait_scatter(o % NRB, (o + 3) % NIB)
        wait_idx(o, o)
        fire_gather(o % NRB, o)
        wait_gather((o + 1) % NRB, (o + 4) % NIB)
        fire_scatter((o + 1) % NRB, (o + 4) % NIB)
        fire_idx(o + 2, (o + 2) % NIB)

    def body(r, carry):
        jb = r * 6
        for o in range(6):
            j = jb + o
            b = o % NRB
            wait_scatter(b, (o + 3) % NIB)
            wait_idx(j, o)
            fire_gather(b, o)
            wait_gather((o + 1) % NRB, (o + 4) % NIB)
            fire_scatter((o + 1) % NRB, (o + 4) % NIB)

            @pl.when(j + 2 < W)
            def _():
                fire_idx(j + 2, (o + 2) % NIB)
        return carry

    lax.fori_loop(1, W // 6, body, 0)

    wait_gather((W - 2) % NRB, (W - 2) % NIB)
    fire_scatter((W - 2) % NRB, (W - 2) % NIB)
    wait_gather((W - 1) % NRB, (W - 1) % NIB)
    fire_scatter((W - 1) % NRB, (W - 1) % NIB)
    for w in (W - 3, W - 2, W - 1):
        wait_scatter(w % NRB, w % NIB)

    plsc.subcore_barrier()

    pltpu.sync_copy(acc.at[pl.ds(base_r, ROWS_PER_TILE)],
                    out_hbm.at[c].at[pl.ds(base_r, ROWS_PER_TILE)])

    @pl.when(s == NS - 1)
    def _():
        pltpu.sync_copy(
            acc.at[pl.ds(NS * ROWS_PER_TILE, ROWS_TAIL)],
            out_hbm.at[c].at[pl.ds(NS * ROWS_PER_TILE, ROWS_TAIL)])


_sc_agg = pl.kernel(
    _sc_agg_body,
    out_type=jax.ShapeDtypeStruct((NC, NA, HALF), jnp.float32),
    mesh=plsc.VectorSubcoreMesh(core_axis_name="c", subcore_axis_name="s"),
    scratch_types=[
        pltpu.VMEM((NIB, 2, K), jnp.int32),
        pltpu.VMEM((NRB, K, HALF), jnp.float32),
        pltpu.VMEM_SHARED((NA, HALF), jnp.float32),
        pltpu.SemaphoreType.DMA((NIB,)),
        pltpu.SemaphoreType.DMA((NRB,)),
        pltpu.SemaphoreType.DMA((NRB,)),
    ],
)


def _tc_weight_body(mem_ref, wihT_ref, bi_ref, bh_ref, wt_ref, bt_ref,
                    out_ref):
    xt = mem_ref[...]
    gi_r = jnp.dot(xt, wihT_ref[0], preferred_element_type=jnp.float32)
    gi_z = jnp.dot(xt, wihT_ref[1], preferred_element_type=jnp.float32)
    gi_n = jnp.dot(xt, wihT_ref[2], preferred_element_type=jnp.float32)
    r = jax.nn.sigmoid(gi_r + bi_ref[0:1, :] + bh_ref[0:1, :])
    z = jax.nn.sigmoid(gi_z + bi_ref[1:2, :] + bh_ref[1:2, :])
    n = jnp.tanh(gi_n + bi_ref[2:3, :] + r * bh_ref[2:3, :])
    u = (1.0 - z) * n
    nw = jnp.sum(wt_ref[...] * u, axis=1, keepdims=True)
    out_ref[...] = nw + bt_ref[...]


def _tc_out_body(h_ref, wT_ref, b_ref, out_ref):
    acc = jnp.dot(h_ref[0], wT_ref[0:HALF, :],
                  preferred_element_type=jnp.float32)
    acc = acc + jnp.dot(h_ref[1], wT_ref[HALF:D, :],
                        preferred_element_type=jnp.float32)
    out_ref[...] = jnp.maximum(acc + b_ref[...], 0.0)


def kernel(x, edge_index, memory, W_ih, W_hh, b_ih, b_hh, W_t, b_t, b_lin):
    del W_hh

    xh = x.reshape(N, NC, HALF).transpose(1, 0, 2)
    xh = jnp.concatenate(
        [xh, jnp.zeros((NC, NPAD, HALF), jnp.float32)], axis=1)

    pad_idx = jnp.tile(N + jnp.arange(NPAD, dtype=jnp.int32), K // NPAD)
    pad_win = jnp.broadcast_to(pad_idx, (NS, 1, K))
    src4 = jnp.concatenate(
        [edge_index[0].reshape(NS, W - 1, K), pad_win], axis=1)
    dst4 = jnp.concatenate(
        [edge_index[1].reshape(NS, W - 1, K), pad_win], axis=1)
    e4 = jnp.stack([src4, dst4], axis=2)

    wihT = W_ih.reshape(3, D, D).transpose(0, 2, 1)
    bi = b_ih.reshape(3, D)
    bh = b_hh.reshape(3, D)
    bt = b_t.reshape(D * D, 1)

    h2 = _sc_agg(xh, e4)

    wt_blk = 8192
    n_wt = (D * D) // wt_blk
    new_w = pl.pallas_call(
        _tc_weight_body,
        grid=(n_wt,),
        in_specs=[
            pl.BlockSpec((1, D), lambda k: (0, 0)),
            pl.BlockSpec((3, D, D), lambda k: (0, 0, 0)),
            pl.BlockSpec((3, D), lambda k: (0, 0)),
            pl.BlockSpec((3, D), lambda k: (0, 0)),
            pl.BlockSpec((wt_blk, D), lambda k: (k, 0)),
            pl.BlockSpec((wt_blk, 1), lambda k: (k, 0)),
        ],
        out_specs=pl.BlockSpec((wt_blk, 1), lambda k: (k, 0)),
        out_shape=jax.ShapeDtypeStruct((D * D, 1), jnp.float32),
    )(memory, wihT, bi, bh, W_t, bt)

    wT = new_w.reshape(D, D).T

    row_blk = 1000
    out = pl.pallas_call(
        _tc_out_body,
        grid=(N // row_blk,),
        in_specs=[
            pl.BlockSpec((NC, row_blk, HALF), lambda i: (0, i, 0)),
            pl.BlockSpec((D, D), lambda i: (0, 0)),
            pl.BlockSpec((1, D), lambda i: (0, 0)),
        ],
        out_specs=pl.BlockSpec((row_blk, D), lambda i: (i, 0)),
        out_shape=jax.ShapeDtypeStruct((N, D), jnp.float32),
    )(h2[:, :N], wT, b_lin.reshape(1, D))
    return out


# speedup vs baseline: 6.6562x; 1.1385x over previous
import jax
import jax.numpy as jnp
from jax import lax
from jax.experimental import pallas as pl
from jax.experimental.pallas import tpu as pltpu
from jax.experimental.pallas import tpu_sc as plsc

N = 10000
E = 160000
D = 256
HALF = 128
NC = 2
NS = 16
K = 80
W = 126
NPAD = 16
NA = N + NPAD
ROWS_PER_TILE = 624
ROWS_TAIL = N - NS * ROWS_PER_TILE
NRB = 3
NIB = 6


def _sc_agg_body(x_hbm, e4_hbm, out_hbm, idx_v, rows_v, acc,
                 sem_i, sem_g, sem_s):
    c = lax.axis_index("c")
    s = lax.axis_index("s")

    table = x_hbm.at[:, pl.ds(c * HALF, HALF)]

    base_r = s * ROWS_PER_TILE
    pltpu.sync_copy(table.at[pl.ds(base_r, ROWS_PER_TILE)],
                    acc.at[pl.ds(base_r, ROWS_PER_TILE)])

    @pl.when(s == NS - 1)
    def _():
        pltpu.sync_copy(
            table.at[pl.ds(NS * ROWS_PER_TILE, ROWS_TAIL)],
            acc.at[pl.ds(NS * ROWS_PER_TILE, ROWS_TAIL)])

    plsc.subcore_barrier()

    e4 = e4_hbm.at[s]

    def fire_idx(j, si):
        return pltpu.async_copy(e4.at[j], idx_v.at[si], sem_i.at[si])

    def wait_idx(j, si):
        pltpu.make_async_copy(e4.at[j], idx_v.at[si], sem_i.at[si]).wait()

    def fire_gather(b, si):
        return pltpu.async_copy(table.at[idx_v.at[si, 0]], rows_v.at[b],
                                sem_g.at[b])

    def wait_gather(b, si):
        pltpu.make_async_copy(table.at[idx_v.at[si, 0]], rows_v.at[b],
                              sem_g.at[b]).wait()

    def fire_scatter(b, si):
        return pltpu.async_copy(rows_v.at[b], acc.at[idx_v.at[si, 1]],
                                sem_s.at[b], add=True)

    def wait_scatter(b, si):
        pltpu.make_async_copy(rows_v.at[b], acc.at[idx_v.at[si, 1]],
                              sem_s.at[b]).wait()

    fire_idx(0, 0)
    fire_idx(1, 1)
    wait_idx(0, 0)
    fire_gather(0, 0)
    fire_idx(2, 2)
    wait_idx(1, 1)
    fire_gather(1, 1)
    fire_idx(3, 3)
    for o in range(2, 6):
        if o >= 3:
            wait_scatter(o % NRB, (o + 3) % NIB)
        wait_idx(o, o)
        fire_gather(o % NRB, o)
        wait_gather((o + 1) % NRB, (o + 4) % NIB)
        fire_scatter((o + 1) % NRB, (o + 4) % NIB)
        fire_idx(o + 2, (o + 2) % NIB)

    def body(r, carry):
        jb = r * 6
        for o in range(6):
            j = jb + o
            b = o % NRB
            wait_scatter(b, (o + 3) % NIB)
            wait_idx(j, o)
            fire_gather(b, o)
            wait_gather((o + 1) % NRB, (o + 4) % NIB)
            fire_scatter((o + 1) % NRB, (o + 4) % NIB)

            @pl.when(j + 2 < W)
            def _():
                fire_idx(j + 2, (o + 2) % NIB)
        return carry

    lax.fori_loop(1, W // 6, body, 0)

    wait_gather((W - 2) % NRB, (W - 2) % NIB)
    fire_scatter((W - 2) % NRB, (W - 2) % NIB)
    wait_gather((W - 1) % NRB, (W - 1) % NIB)
    fire_scatter((W - 1) % NRB, (W - 1) % NIB)
    for w in (W - 3, W - 2, W - 1):
        wait_scatter(w % NRB, w % NIB)

    plsc.subcore_barrier()

    pltpu.sync_copy(acc.at[pl.ds(base_r, ROWS_PER_TILE)],
                    out_hbm.at[c].at[pl.ds(base_r, ROWS_PER_TILE)])

    @pl.when(s == NS - 1)
    def _():
        pltpu.sync_copy(
            acc.at[pl.ds(NS * ROWS_PER_TILE, ROWS_TAIL)],
            out_hbm.at[c].at[pl.ds(NS * ROWS_PER_TILE, ROWS_TAIL)])


_sc_agg = pl.kernel(
    _sc_agg_body,
    out_type=jax.ShapeDtypeStruct((NC, N, HALF), jnp.float32),
    mesh=plsc.VectorSubcoreMesh(core_axis_name="c", subcore_axis_name="s"),
    scratch_types=[
        pltpu.VMEM((NIB, 2, K), jnp.int32),
        pltpu.VMEM((NRB, K, HALF), jnp.float32),
        pltpu.VMEM_SHARED((NA, HALF), jnp.float32),
        pltpu.SemaphoreType.DMA((NIB,)),
        pltpu.SemaphoreType.DMA((NRB,)),
        pltpu.SemaphoreType.DMA((NRB,)),
    ],
)


def _tc_weight_body(mem_ref, wihT_ref, bi_ref, bh_ref, wt_ref, bt_ref,
                    out_ref):
    xt = mem_ref[...]
    gi_r = jnp.dot(xt, wihT_ref[0], preferred_element_type=jnp.float32)
    gi_z = jnp.dot(xt, wihT_ref[1], preferred_element_type=jnp.float32)
    gi_n = jnp.dot(xt, wihT_ref[2], preferred_element_type=jnp.float32)
    r = jax.nn.sigmoid(gi_r + bi_ref[0:1, :] + bh_ref[0:1, :])
    z = jax.nn.sigmoid(gi_z + bi_ref[1:2, :] + bh_ref[1:2, :])
    n = jnp.tanh(gi_n + bi_ref[2:3, :] + r * bh_ref[2:3, :])
    u = (1.0 - z) * n
    nw = jnp.sum(wt_ref[...] * u, axis=1, keepdims=True)
    out_ref[...] = nw + bt_ref[...]


def _tc_out_body(h_ref, wT_ref, b_ref, out_ref):
    acc = jnp.dot(h_ref[0], wT_ref[0:HALF, :],
                  preferred_element_type=jnp.float32)
    acc = acc + jnp.dot(h_ref[1], wT_ref[HALF:D, :],
                        preferred_element_type=jnp.float32)
    out_ref[...] = jnp.maximum(acc + b_ref[...], 0.0)


def kernel(x, edge_index, memory, W_ih, W_hh, b_ih, b_hh, W_t, b_t, b_lin):
    del W_hh

    pad_src = (jnp.arange(NS, dtype=jnp.int32)[:, None, None] * 617
               + jnp.arange(K, dtype=jnp.int32) * 7) % N
    pad_dst = jnp.broadcast_to(
        N + jnp.tile(jnp.arange(NPAD, dtype=jnp.int32), K // NPAD),
        (NS, 1, K))
    src4 = jnp.concatenate(
        [edge_index[0].reshape(NS, W - 1, K), pad_src], axis=1)
    dst4 = jnp.concatenate(
        [edge_index[1].reshape(NS, W - 1, K), pad_dst], axis=1)
    e4 = jnp.stack([src4, dst4], axis=2)

    wihT = W_ih.reshape(3, D, D).transpose(0, 2, 1)
    bi = b_ih.reshape(3, D)
    bh = b_hh.reshape(3, D)
    bt = b_t.reshape(D * D, 1)

    h2 = _sc_agg(x, e4)

    wt_blk = 8192
    n_wt = (D * D) // wt_blk
    new_w = pl.pallas_call(
        _tc_weight_body,
        grid=(n_wt,),
        in_specs=[
            pl.BlockSpec((1, D), lambda k: (0, 0)),
            pl.BlockSpec((3, D, D), lambda k: (0, 0, 0)),
            pl.BlockSpec((3, D), lambda k: (0, 0)),
            pl.BlockSpec((3, D), lambda k: (0, 0)),
            pl.BlockSpec((wt_blk, D), lambda k: (k, 0)),
            pl.BlockSpec((wt_blk, 1), lambda k: (k, 0)),
        ],
        out_specs=pl.BlockSpec((wt_blk, 1), lambda k: (k, 0)),
        out_shape=jax.ShapeDtypeStruct((D * D, 1), jnp.float32),
    )(memory, wihT, bi, bh, W_t, bt)

    wT = new_w.reshape(D, D).T

    row_blk = 1000
    out = pl.pallas_call(
        _tc_out_body,
        grid=(N // row_blk,),
        in_specs=[
            pl.BlockSpec((NC, row_blk, HALF), lambda i: (0, i, 0)),
            pl.BlockSpec((D, D), lambda i: (0, 0)),
            pl.BlockSpec((1, D), lambda i: (0, 0)),
        ],
        out_specs=pl.BlockSpec((row_blk, D), lambda i: (i, 0)),
        out_shape=jax.ShapeDtypeStruct((N, D), jnp.float32),
    )(h2, wT, b_lin.reshape(1, D))
    return out
